# baseline (speedup 1.0000x reference)
import functools

import jax
import jax.numpy as jnp
from jax import lax
from jax.experimental import pallas as pl
from jax.experimental.pallas import tpu as pltpu
from jax.experimental.pallas import tpu_sc as plsc

VOCAB_ROWS = 1000000
D = 64
DP = 128
NPAIR = VOCAB_ROWS // 2
B = 4096 * 50
NC = 2
NS = 16
NW = NC * NS
BPW = B // NW
CHUNK = 64
K = 5
ROWS = K * CHUNK
STEPS = BPW // ROWS

NBLK = VOCAB_ROWS // 128
BLK_PER_W = -(-NBLK // NW)
TAIL = VOCAB_ROWS - NBLK * 128

_mesh = plsc.VectorSubcoreMesh(core_axis_name="c", subcore_axis_name="s")


@functools.partial(
    pl.kernel,
    mesh=_mesh,
    compiler_params=pltpu.CompilerParams(
        use_tc_tiling_on_sc=True, needs_layout_passes=False),
    out_type=jax.ShapeDtypeStruct((NPAIR, DP), jnp.float32),
    scratch_types=[
        pltpu.VMEM((D, 128), jnp.float32),
        pltpu.VMEM((D, DP), jnp.float32),
        pltpu.VMEM((TAIL // 2, DP), jnp.float32),
    ],
)
def _transpose(ttr_hbm, tailp_hbm, pairs_hbm, bin_, bout, btail):
    wid = lax.axis_index("s") * NC + lax.axis_index("c")
    lanes = lax.iota(jnp.int32, 16)

    @pl.when(wid == 0)
    def _():
        pltpu.sync_copy(tailp_hbm, btail)
        pltpu.sync_copy(btail, pairs_hbm.at[pl.ds(NBLK * D, TAIL // 2)])

    def blk(k):
        u = wid + NW * k

        @pl.when(u < NBLK)
        def _():
            c0 = u * 128
            pltpu.sync_copy(ttr_hbm.at[:, pl.ds(c0, 128)], bin_)

            def rows4(p0):
                for dp in range(4):
                    p = p0 + dp
                    for h in range(2):
                        col = jnp.full((16,), 2 * p + h, jnp.int32)
                        for q in range(D // 16):
                            vec = plsc.load_gather(
                                bin_, [q * 16 + lanes, col])
                            bout[p, pl.ds(h * D + q * 16, 16)] = vec

            pl.loop(0, D, step=4)(rows4)
            pltpu.sync_copy(bout, pairs_hbm.at[pl.ds(u * D, D)])

    pl.loop(0, BLK_PER_W)(blk)


@functools.partial(
    pl.kernel,
    mesh=_mesh,
    compiler_params=pltpu.CompilerParams(use_tc_tiling_on_sc=False),
    out_type=jax.ShapeDtypeStruct((B, D), jnp.float32),
    scratch_types=[
        pltpu.VMEM((BPW,), jnp.int32),
        pltpu.VMEM((BPW,), jnp.int32),
        pltpu.VMEM((ROWS, DP), jnp.float32),
        pltpu.VMEM((ROWS, DP), jnp.float32),
        pltpu.SemaphoreType.DMA,
        pltpu.SemaphoreType.DMA,
        pltpu.SemaphoreType.DMA,
        pltpu.SemaphoreType.DMA,
    ],
)
def _embed(idx_hbm, pairs_hbm, out_hbm, idx_v, pidx_v, rows0, rows1,
           g0, g1, w0, w1):
    wid = lax.axis_index("s") * NC + lax.axis_index("c")
    base = wid * BPW
    pltpu.sync_copy(idx_hbm.at[wid], idx_v)

    def mkpair(i):
        v = idx_v[pl.ds(i * 16, 16)]
        pidx_v[pl.ds(i * 16, 16)] = lax.shift_right_logical(v, 1)

    pl.loop(0, BPW // 16)(mkpair)

    bufs = ((rows0, g0, w0), (rows1, g1, w1))

    def fire(t, rows, gsem):
        handles = []
        for j in range(K):
            handles.append(pltpu.async_copy(
                pairs_hbm.at[pidx_v.at[pl.ds((t * K + j) * CHUNK, CHUNK)]],
                rows.at[pl.ds(j * CHUNK, CHUNK)],
                gsem,
            ))
        return handles

    def fix(t, rows):
        def fix16(g):
            vs = idx_v[pl.ds(t * ROWS + g * 16, 16)]
            for j in range(16):
                @pl.when(vs[j] % 2 == 1)
                def _(j=j):
                    i = g * 16 + j
                    for q in range(D // 16):
                        rows[i, pl.ds(q * 16, 16)] = (
                            rows[i, pl.ds(D + q * 16, 16)])

        pl.loop(0, ROWS // 16)(fix16)

    def step2(s):
        all_handles = []
        for b, (rows, gsem, wsem) in enumerate(bufs):
            t = s + b

            @pl.when(t >= 2)
            def _():
                pltpu.make_async_copy(
                    rows.at[:, :D],
                    out_hbm.at[pl.ds(base + (t - 2) * ROWS, ROWS)],
                    wsem,
                ).wait()

            all_handles.append(fire(t, rows, gsem))

        for b, (rows, gsem, wsem) in enumerate(bufs):
            t = s + b
            for h in all_handles[b]:
                h.wait()
            fix(t, rows)
            pltpu.async_copy(
                rows.at[:, :D],
                out_hbm.at[pl.ds(base + t * ROWS, ROWS)],
                wsem,
            )

    pl.loop(0, STEPS, step=2)(step2)

    for b, (rows, gsem, wsem) in enumerate(bufs):
        t = STEPS - 2 + b
        pltpu.make_async_copy(
            rows.at[:, :D],
            out_hbm.at[pl.ds(base + t * ROWS, ROWS)],
            wsem,
        ).wait()


def kernel(x, table):
    idx = x.astype(jnp.int32).reshape(NW, BPW)
    ttr = table.T
    tailp = table[NBLK * 128:].reshape(TAIL // 2, DP)
    pairs = _transpose(ttr, tailp)
    out = _embed(idx, pairs)
    return out.reshape(x.shape[0], x.shape[1], D)


# speedup vs baseline: 1.1209x; 1.1209x over previous
import functools

import jax
import jax.numpy as jnp
from jax import lax
from jax.experimental import pallas as pl
from jax.experimental.pallas import tpu as pltpu
from jax.experimental.pallas import tpu_sc as plsc

VOCAB_ROWS = 1000000
D = 64
DP = 128
NPAIR = VOCAB_ROWS // 2
B = 4096 * 50
NC = 2
NS = 16
NW = NC * NS
BPW = B // NW
CHUNK = 64
K = 5
ROWS = K * CHUNK
STEPS = BPW // ROWS

NBLK = VOCAB_ROWS // 128
BLK_PER_W = 246
TAIL = VOCAB_ROWS - NBLK * 128

_mesh = plsc.VectorSubcoreMesh(core_axis_name="c", subcore_axis_name="s")


@functools.partial(
    pl.kernel,
    mesh=_mesh,
    compiler_params=pltpu.CompilerParams(
        use_tc_tiling_on_sc=True, needs_layout_passes=False),
    out_type=jax.ShapeDtypeStruct((NPAIR, DP), jnp.float32),
    scratch_types=[
        pltpu.VMEM((D, 128), jnp.float32),
        pltpu.VMEM((D, 128), jnp.float32),
        pltpu.VMEM((D, DP), jnp.float32),
        pltpu.VMEM((D, DP), jnp.float32),
        pltpu.VMEM((TAIL // 2, DP), jnp.float32),
        pltpu.SemaphoreType.DMA,
        pltpu.SemaphoreType.DMA,
        pltpu.SemaphoreType.DMA,
        pltpu.SemaphoreType.DMA,
    ],
)
def _transpose(ttr_hbm, tailp_hbm, pairs_hbm, bin0, bin1, bout0, bout1,
               btail, gi0, gi1, wo0, wo1):
    wid = lax.axis_index("s") * NC + lax.axis_index("c")
    lanes = lax.iota(jnp.int32, 16)

    @pl.when(wid == 0)
    def _():
        pltpu.sync_copy(tailp_hbm, btail)
        pltpu.sync_copy(btail, pairs_hbm.at[pl.ds(NBLK * D, TAIL // 2)])

    bufs = ((bin0, bout0, gi0, wo0), (bin1, bout1, gi1, wo1))

    def transform(bin_, bout):
        def rows8(p0):
            for dp in range(8):
                p = p0 + dp
                for h in range(2):
                    col = jnp.full((16,), 2 * p + h, jnp.int32)
                    for q in range(D // 16):
                        vec = plsc.load_gather(bin_, [q * 16 + lanes, col])
                        bout[p, pl.ds(h * D + q * 16, 16)] = vec

        pl.loop(0, D, step=8)(rows8)

    def body(j):
        for b, (bin_, bout, gi, wo) in enumerate(bufs):
            t = 2 * j + b
            u = wid + NW * t

            @pl.when(u < NBLK)
            def _(bin_=bin_, bout=bout, gi=gi, wo=wo, t=t, u=u):
                @pl.when(t >= 2)
                def _():
                    pltpu.make_async_copy(
                        bout, pairs_hbm.at[pl.ds((u - 2 * NW) * D, D)], wo
                    ).wait()

                pltpu.async_copy(
                    ttr_hbm.at[:, pl.ds(u * 128, 128)], bin_, gi)

        for b, (bin_, bout, gi, wo) in enumerate(bufs):
            t = 2 * j + b
            u = wid + NW * t

            @pl.when(u < NBLK)
            def _(bin_=bin_, bout=bout, gi=gi, wo=wo, u=u):
                pltpu.make_async_copy(
                    ttr_hbm.at[:, pl.ds(u * 128, 128)], bin_, gi).wait()
                transform(bin_, bout)
                pltpu.async_copy(bout, pairs_hbm.at[pl.ds(u * D, D)], wo)

    pl.loop(0, BLK_PER_W // 2)(body)

    for t in (BLK_PER_W - 4, BLK_PER_W - 3, BLK_PER_W - 2):
        u = wid + NW * t
        bin_, bout, gi, wo = bufs[t % 2]

        @pl.when((u < NBLK) & (u + 2 * NW >= NBLK))
        def _(bout=bout, wo=wo, u=u):
            pltpu.make_async_copy(
                bout, pairs_hbm.at[pl.ds(u * D, D)], wo).wait()


@functools.partial(
    pl.kernel,
    mesh=_mesh,
    compiler_params=pltpu.CompilerParams(use_tc_tiling_on_sc=False),
    out_type=jax.ShapeDtypeStruct((B, D), jnp.float32),
    scratch_types=[
        pltpu.VMEM((BPW,), jnp.int32),
        pltpu.VMEM((BPW,), jnp.int32),
        pltpu.VMEM((ROWS, DP), jnp.float32),
        pltpu.VMEM((ROWS, DP), jnp.float32),
        pltpu.SemaphoreType.DMA,
        pltpu.SemaphoreType.DMA,
        pltpu.SemaphoreType.DMA,
        pltpu.SemaphoreType.DMA,
    ],
)
def _embed(idx_hbm, pairs_hbm, out_hbm, idx_v, pidx_v, rows0, rows1,
           g0, g1, w0, w1):
    wid = lax.axis_index("s") * NC + lax.axis_index("c")
    base = wid * BPW
    pltpu.sync_copy(idx_hbm.at[wid], idx_v)

    def mkpair(i):
        v = idx_v[pl.ds(i * 16, 16)]
        pidx_v[pl.ds(i * 16, 16)] = lax.shift_right_logical(v, 1)

    pl.loop(0, BPW // 16)(mkpair)

    bufs = ((rows0, g0, w0), (rows1, g1, w1))

    def fire(t, rows, gsem):
        handles = []
        for j in range(K):
            handles.append(pltpu.async_copy(
                pairs_hbm.at[pidx_v.at[pl.ds((t * K + j) * CHUNK, CHUNK)]],
                rows.at[pl.ds(j * CHUNK, CHUNK)],
                gsem,
            ))
        return handles

    def fix(t, rows):
        def fix16(g):
            vs = idx_v[pl.ds(t * ROWS + g * 16, 16)]
            for j in range(16):
                @pl.when(vs[j] % 2 == 1)
                def _(j=j):
                    i = g * 16 + j
                    for q in range(D // 16):
                        rows[i, pl.ds(q * 16, 16)] = (
                            rows[i, pl.ds(D + q * 16, 16)])

        pl.loop(0, ROWS // 16)(fix16)

    def step2(s):
        all_handles = []
        for b, (rows, gsem, wsem) in enumerate(bufs):
            t = s + b

            @pl.when(t >= 2)
            def _():
                pltpu.make_async_copy(
                    rows.at[:, :D],
                    out_hbm.at[pl.ds(base + (t - 2) * ROWS, ROWS)],
                    wsem,
                ).wait()

            all_handles.append(fire(t, rows, gsem))

        for b, (rows, gsem, wsem) in enumerate(bufs):
            t = s + b
            for h in all_handles[b]:
                h.wait()
            fix(t, rows)
            pltpu.async_copy(
                rows.at[:, :D],
                out_hbm.at[pl.ds(base + t * ROWS, ROWS)],
                wsem,
            )

    pl.loop(0, STEPS, step=2)(step2)

    for b, (rows, gsem, wsem) in enumerate(bufs):
        t = STEPS - 2 + b
        pltpu.make_async_copy(
            rows.at[:, :D],
            out_hbm.at[pl.ds(base + t * ROWS, ROWS)],
            wsem,
        ).wait()


def kernel(x, table):
    idx = x.astype(jnp.int32).reshape(NW, BPW)
    ttr = table.T
    tailp = table[NBLK * 128:].reshape(TAIL // 2, DP)
    pairs = _transpose(ttr, tailp)
    out = _embed(idx, pairs)
    return out.reshape(x.shape[0], x.shape[1], D)


# speedup vs baseline: 2.6991x; 2.4079x over previous
import functools

import jax
import jax.numpy as jnp
from jax import lax
from jax.experimental import pallas as pl
from jax.experimental.pallas import tpu as pltpu
from jax.experimental.pallas import tpu_sc as plsc

D = 64
DP = 128
B = 4096 * 50
NC = 2
NS = 16
NW = NC * NS
BPW = B // NW
CHUNK = 64
K = 5
ROWS = K * CHUNK
STEPS = BPW // ROWS

_mesh = plsc.VectorSubcoreMesh(core_axis_name="c", subcore_axis_name="s")


@functools.partial(
    pl.kernel,
    mesh=_mesh,
    compiler_params=pltpu.CompilerParams(use_tc_tiling_on_sc=False),
    out_type=jax.ShapeDtypeStruct((B, DP), jnp.float32),
    scratch_types=[
        pltpu.VMEM((STEPS * K, CHUNK), jnp.int32),
        pltpu.VMEM((ROWS, DP), jnp.float32),
        pltpu.VMEM((ROWS, DP), jnp.float32),
        pltpu.SemaphoreType.DMA,
        pltpu.SemaphoreType.DMA,
        pltpu.SemaphoreType.DMA,
        pltpu.SemaphoreType.DMA,
    ],
)
def _embed(idx_hbm, table_hbm, out_hbm, idx_v, rows0, rows1, g0, g1, w0, w1):
    wid = lax.axis_index("s") * NC + lax.axis_index("c")
    base = wid * BPW
    pltpu.sync_copy(idx_hbm.at[wid], idx_v)

    bufs = ((rows0, g0, w0), (rows1, g1, w1))

    def fire(t, rows, gsem):
        handles = []
        for j in range(K):
            handles.append(pltpu.async_copy(
                table_hbm.at[idx_v.at[t * K + j]],
                rows.at[pl.ds(j * CHUNK, CHUNK)],
                gsem,
            ))
        return handles

    def step2(s):
        all_handles = []
        for b, (rows, gsem, wsem) in enumerate(bufs):
            t = s + b

            @pl.when(t >= 2)
            def _():
                pltpu.make_async_copy(
                    rows, out_hbm.at[pl.ds(base + (t - 2) * ROWS, ROWS)], wsem
                ).wait()

            all_handles.append(fire(t, rows, gsem))

        for b, (rows, gsem, wsem) in enumerate(bufs):
            t = s + b
            for h in all_handles[b]:
                h.wait()
            pltpu.async_copy(
                rows, out_hbm.at[pl.ds(base + t * ROWS, ROWS)], wsem
            )

    pl.loop(0, STEPS, step=2)(step2)

    for b, (rows, gsem, wsem) in enumerate(bufs):
        t = STEPS - 2 + b
        pltpu.make_async_copy(
            rows, out_hbm.at[pl.ds(base + t * ROWS, ROWS)], wsem
        ).wait()


def kernel(x, table):
    idx = x.astype(jnp.int32).reshape(NW, STEPS * K, CHUNK)
    tpad = jnp.pad(table, ((0, 0), (0, DP - D)))
    out = _embed(idx, tpad)
    return out.reshape(x.shape[0], x.shape[1], DP)[:, :, :D]


# speedup vs baseline: 2.7856x; 1.0321x over previous
import functools

import jax
import jax.numpy as jnp
from jax import lax
from jax.experimental import pallas as pl
from jax.experimental.pallas import tpu as pltpu
from jax.experimental.pallas import tpu_sc as plsc

D = 64
DP = 128
B = 4096 * 50
NC = 2
NS = 16
NW = NC * NS
BPW = B // NW
CHUNK = 64
K = 5
ROWS = K * CHUNK
STEPS = BPW // ROWS

_mesh = plsc.VectorSubcoreMesh(core_axis_name="c", subcore_axis_name="s")


@functools.partial(
    pl.kernel,
    mesh=_mesh,
    compiler_params=pltpu.CompilerParams(use_tc_tiling_on_sc=False),
    out_type=jax.ShapeDtypeStruct((B, D), jnp.float32),
    scratch_types=[
        pltpu.VMEM((STEPS * K, CHUNK), jnp.int32),
        pltpu.VMEM((ROWS, DP), jnp.float32),
        pltpu.VMEM((ROWS, DP), jnp.float32),
        pltpu.SemaphoreType.DMA,
        pltpu.SemaphoreType.DMA,
        pltpu.SemaphoreType.DMA,
        pltpu.SemaphoreType.DMA,
    ],
)
def _embed(idx_hbm, table_hbm, out_hbm, idx_v, rows0, rows1, g0, g1, w0, w1):
    wid = lax.axis_index("s") * NC + lax.axis_index("c")
    base = wid * BPW
    pltpu.sync_copy(idx_hbm.at[wid], idx_v)

    bufs = ((rows0, g0, w0), (rows1, g1, w1))

    def fire(t, rows, gsem):
        handles = []
        for j in range(K):
            handles.append(pltpu.async_copy(
                table_hbm.at[idx_v.at[t * K + j]],
                rows.at[pl.ds(j * CHUNK, CHUNK)],
                gsem,
            ))
        return handles

    def step2(s):
        all_handles = []
        for b, (rows, gsem, wsem) in enumerate(bufs):
            t = s + b

            @pl.when(t >= 2)
            def _():
                pltpu.make_async_copy(
                    rows.at[:, :D],
                    out_hbm.at[pl.ds(base + (t - 2) * ROWS, ROWS)],
                    wsem,
                ).wait()

            all_handles.append(fire(t, rows, gsem))

        for b, (rows, gsem, wsem) in enumerate(bufs):
            t = s + b
            for h in all_handles[b]:
                h.wait()
            pltpu.async_copy(
                rows.at[:, :D],
                out_hbm.at[pl.ds(base + t * ROWS, ROWS)],
                wsem,
            )

    pl.loop(0, STEPS, step=2)(step2)

    for b, (rows, gsem, wsem) in enumerate(bufs):
        t = STEPS - 2 + b
        pltpu.make_async_copy(
            rows.at[:, :D],
            out_hbm.at[pl.ds(base + t * ROWS, ROWS)],
            wsem,
        ).wait()


def kernel(x, table):
    idx = x.astype(jnp.int32).reshape(NW, STEPS * K, CHUNK)
    tpad = jnp.pad(table, ((0, 0), (0, DP - D)))
    out = _embed(idx, tpad)
    return out.reshape(x.shape[0], x.shape[1], D)
